# baseline (speedup 1.0000x reference)
import functools

import jax
import jax.numpy as jnp
from jax import lax
from jax.experimental import pallas as pl
from jax.experimental.pallas import tpu as pltpu
from jax.experimental.pallas import tpu_sc as plsc

N = 8192
D_IN = 1024
D = 512
DA = 128
K_MASK = 819
R = 512
T = N // R

NT = 16
CH = N // NT
NV = CH // 16


def _stage_a(x_ref, w1_ref, b1_ref, va_ref, ba_ref, wa_ref, bwa_ref,
             feat_ref, s_ref):
    xt = x_ref[...]
    feat = jnp.maximum(
        lax.dot_general(xt, w1_ref[...], (((1,), (0,)), ((), ()))) +
        b1_ref[...], 0.0)
    feat_ref[...] = feat.astype(jnp.bfloat16)
    h = jax.nn.gelu(
        lax.dot_general(feat, va_ref[...], (((1,), (0,)), ((), ()))) +
        ba_ref[...])
    s = lax.dot_general(
        wa_ref[...], h, (((0,), (1,)), ((), ()))) + bwa_ref[0, 0]
    s_ref[...] = s.reshape(s_ref.shape)


def _stage_b(scores_hbm, w_hbm, s_v, m_v, h2d, gh, ghf, e_v, t16f,
             sh_hist, sh_f):
    cid = lax.axis_index("c")
    sid = lax.axis_index("s")

    @pl.when(cid == 0)
    def _work():
        lane = lax.iota(jnp.int32, 16)
        ones_i = jnp.ones((16,), jnp.int32)

        pltpu.sync_copy(scores_hbm.at[pl.ds(sid * CH, CH)], s_v)

        for j in range(NV):
            b = lax.bitcast_convert_type(s_v[pl.ds(j * 16, 16)], jnp.int32)
            m_v[pl.ds(j * 16, 16)] = b ^ ((b >> 31) & jnp.int32(0x7FFFFFFF))

        k_rem = jnp.int32(K_MASK)
        pval = jnp.int32(0)
        for r in range(4):
            shift = 24 - 8 * r

            def _zero(j, _):
                for k in range(8):
                    h2d[pl.ds(j * 128 + k * 16, 16)] = jnp.zeros(
                        (16,), jnp.int32)
                return 0
            lax.fori_loop(0, 32, _zero, 0)

            def _hist(j, _):
                m = m_v[pl.ds(j * 16, 16)]
                if r == 0:
                    byte = (m >> 24) + 128
                    plsc.addupdate_scatter(h2d, [lane * 256 + byte], ones_i)
                else:
                    byte = (m >> shift) & 255
                    pm = (m >> (shift + 8)) == pval
                    plsc.addupdate_scatter(
                        h2d, [lane * 256 + byte], ones_i, mask=pm)
                return 0
            lax.fori_loop(0, NV, _hist, 0)

            def _lmerge(j, _):
                a = h2d[pl.ds(j * 16, 16)]
                for l in range(1, 16):
                    a = a + h2d[pl.ds(l * 256 + j * 16, 16)]
                gh[pl.ds(j * 16, 16)] = a
                return 0
            lax.fori_loop(0, 16, _lmerge, 0)

            half = (r % 2) * 4096
            pltpu.sync_copy(gh, sh_hist.at[pl.ds(half + sid * 256, 256)])
            plsc.subcore_barrier()
            pltpu.sync_copy(sh_hist.at[pl.ds(half, 4096)], h2d)
            lax.fori_loop(0, 16, _lmerge, 0)

            def _scan(i, c):
                carry, best = c
                ch = 15 - i
                chunk = gh[pl.ds(ch * 16, 16)]
                suff = jnp.flip(plsc.cumsum(jnp.flip(chunk, 0)), 0) + carry
                bidx = ch * 16 + lane
                cand = jnp.max(jnp.where(suff >= k_rem, bidx, -1))
                return (carry + jnp.sum(chunk), jnp.maximum(best, cand))
            _, bstar = lax.fori_loop(
                0, 16, _scan, (jnp.int32(0), jnp.int32(-1)))

            def _pick(i, c):
                carry, cge, hb = c
                ch = 15 - i
                chunk = gh[pl.ds(ch * 16, 16)]
                suff = jnp.flip(plsc.cumsum(jnp.flip(chunk, 0)), 0) + carry
                sel = (ch * 16 + lane) == bstar
                cge = jnp.maximum(cge, jnp.max(jnp.where(sel, suff, -1)))
                hb = jnp.maximum(hb, jnp.max(jnp.where(sel, chunk, -1)))
                return (carry + jnp.sum(chunk), cge, hb)
            _, cnt_ge, hist_b = lax.fori_loop(
                0, 16, _pick, (jnp.int32(0), jnp.int32(-1), jnp.int32(-1)))

            k_rem = k_rem - (cnt_ge - hist_b)
            pval = (bstar - 128) if r == 0 else ((pval << 8) | bstar)

        vstar = pval
        t_ties = k_rem

        counts = plsc.load_gather(h2d, [lane * 256 + (vstar & 255)])
        n_eq = jnp.sum(counts)
        keep_cnt = n_eq - t_ties
        pref = plsc.cumsum(counts) - counts
        quota = keep_cnt - jnp.sum(jnp.where(lane == sid, pref, 0))

        def _mark(j, c):
            run, mx = c
            m = m_v[pl.ds(j * 16, 16)]
            s = s_v[pl.ds(j * 16, 16)]
            eq = m == vstar
            eqi = eq.astype(jnp.int32)
            rank = plsc.cumsum(eqi) + run
            keep = (m < vstar) | (eq & (rank <= quota))
            m_v[pl.ds(j * 16, 16)] = keep.astype(jnp.int32)
            mx = jnp.maximum(mx, jnp.where(keep, s, -jnp.inf))
            return (run + jnp.sum(eqi), mx)
        _, mxv = lax.fori_loop(
            0, NV, _mark, (jnp.int32(0), jnp.full((16,), -jnp.inf)))
        mx_t = jnp.max(mxv)

        msafe = jnp.maximum(mx_t, jnp.float32(-1e30))

        def _lsum(j, a):
            kp = m_v[pl.ds(j * 16, 16)] == 1
            return a + jnp.where(
                kp, jnp.exp(s_v[pl.ds(j * 16, 16)] - msafe), 0.0)
        s_t = jnp.sum(lax.fori_loop(
            0, NV, _lsum, jnp.zeros((16,), jnp.float32)))

        t16f[...] = jnp.zeros((16,), jnp.float32) + mx_t
        pltpu.sync_copy(t16f, sh_f.at[pl.ds(sid * 16, 16)])
        t16f[...] = jnp.zeros((16,), jnp.float32) + s_t
        pltpu.sync_copy(t16f, sh_f.at[pl.ds(256 + sid * 16, 16)])
        plsc.subcore_barrier()
        pltpu.sync_copy(sh_f, ghf)
        mxs = plsc.load_gather(ghf, [lane * 16])
        sms = plsc.load_gather(ghf, [lane * 16 + 256])
        gmax = jnp.max(mxs)
        z = jnp.sum(sms * jnp.exp(mxs - gmax))

        def _fin(j, _):
            kp = m_v[pl.ds(j * 16, 16)] == 1
            e_v[pl.ds(j * 16, 16)] = jnp.where(
                kp, jnp.exp(s_v[pl.ds(j * 16, 16)] - gmax) / z, 0.0)
            return 0
        lax.fori_loop(0, NV, _fin, 0)
        pltpu.sync_copy(e_v, w_hbm.at[pl.ds(sid * CH, CH)])


_select_sc = functools.partial(
    pl.kernel,
    out_type=jax.ShapeDtypeStruct((N,), jnp.float32),
    mesh=plsc.VectorSubcoreMesh(core_axis_name="c", subcore_axis_name="s",
                                num_cores=2, num_subcores=16),
    compiler_params=pltpu.CompilerParams(needs_layout_passes=False),
    scratch_types=[
        pltpu.VMEM((CH,), jnp.float32),
        pltpu.VMEM((CH,), jnp.int32),
        pltpu.VMEM((4096,), jnp.int32),
        pltpu.VMEM((256,), jnp.int32),
        pltpu.VMEM((512,), jnp.float32),
        pltpu.VMEM((CH,), jnp.float32),
        pltpu.VMEM((16,), jnp.float32),
        pltpu.VMEM_SHARED((8192,), jnp.int32),
        pltpu.VMEM_SHARED((512,), jnp.float32),
    ],
)(_stage_b)


def _stage_c(w_ref, feat_ref, wp_ref, bp_ref, out_ref, acc_ref):
    i = pl.program_id(0)

    @pl.when(i == 0)
    def _init():
        acc_ref[...] = jnp.zeros((1, D), jnp.float32)

    wv = w_ref[...].reshape(1, w_ref.shape[0]).astype(jnp.bfloat16)
    acc_ref[...] += lax.dot_general(
        wv, feat_ref[...],
        (((1,), (0,)), ((), ())), preferred_element_type=jnp.float32)

    @pl.when(i == pl.num_programs(0) - 1)
    def _fin():
        out_ref[...] = lax.dot_general(
            acc_ref[...], wp_ref[...], (((1,), (0,)), ((), ()))) + bp_ref[...]


@jax.jit
def kernel(x, W1, b1, Va, ba, wa, bwa, Wp, bp):
    x2 = x.reshape(N, D_IN)
    feat, scores = pl.pallas_call(
        _stage_a,
        grid=(T,),
        in_specs=[
            pl.BlockSpec((R, D_IN), lambda i: (i, 0)),
            pl.BlockSpec((D_IN, D), lambda i: (0, 0)),
            pl.BlockSpec((1, D), lambda i: (0, 0)),
            pl.BlockSpec((D, DA), lambda i: (0, 0)),
            pl.BlockSpec((1, DA), lambda i: (0, 0)),
            pl.BlockSpec((DA, 1), lambda i: (0, 0)),
            pl.BlockSpec((1, 1), lambda i: (0, 0)),
        ],
        out_specs=[
            pl.BlockSpec((R, D), lambda i: (i, 0)),
            pl.BlockSpec((R,), lambda i: (i,)),
        ],
        out_shape=[
            jax.ShapeDtypeStruct((N, D), jnp.bfloat16),
            jax.ShapeDtypeStruct((N,), jnp.float32),
        ],
    )(x2, W1, b1.reshape(1, D), Va, ba.reshape(1, DA), wa,
      bwa.reshape(1, 1))

    w = _select_sc(scores)

    out = pl.pallas_call(
        _stage_c,
        grid=(8,),
        in_specs=[
            pl.BlockSpec((N // 8,), lambda i: (i,)),
            pl.BlockSpec((N // 8, D), lambda i: (i, 0)),
            pl.BlockSpec((D, 2), lambda i: (0, 0)),
            pl.BlockSpec((1, 2), lambda i: (0, 0)),
        ],
        out_specs=pl.BlockSpec((1, 2), lambda i: (0, 0)),
        out_shape=jax.ShapeDtypeStruct((1, 2), jnp.float32),
        scratch_shapes=[pltpu.VMEM((1, D), jnp.float32)],
    )(w, feat, Wp, bp.reshape(1, 2))
    return out


# speedup vs baseline: 1.0858x; 1.0858x over previous
import functools

import jax
import jax.numpy as jnp
from jax import lax
from jax.experimental import pallas as pl
from jax.experimental.pallas import tpu as pltpu
from jax.experimental.pallas import tpu_sc as plsc

N = 8192
D_IN = 1024
D = 512
DA = 128
K_MASK = 819
R = 1024
T = N // R

NT = 16
CH = N // NT
NV = CH // 16


def _stage_a(x_ref, w1_ref, b1_ref, va_ref, ba_ref, wa_ref, bwa_ref,
             feat_ref, s_ref):
    xt = x_ref[...]
    feat = jnp.maximum(
        lax.dot_general(xt, w1_ref[...], (((1,), (0,)), ((), ()))) +
        b1_ref[...], 0.0)
    feat_ref[...] = feat.astype(jnp.bfloat16)
    h = jax.nn.gelu(
        lax.dot_general(feat, va_ref[...], (((1,), (0,)), ((), ()))) +
        ba_ref[...])
    s = lax.dot_general(
        wa_ref[...], h, (((0,), (1,)), ((), ()))) + bwa_ref[0, 0]
    s_ref[...] = s.reshape(s_ref.shape)


def _stage_b(scores_hbm, w_hbm, s_v, m_v, h2d, gh, ghf, e_v, t16f,
             sh_hist, sh_f):
    cid = lax.axis_index("c")
    sid = lax.axis_index("s")

    @pl.when(cid == 0)
    def _work():
        lane = lax.iota(jnp.int32, 16)
        ones_i = jnp.ones((16,), jnp.int32)

        pltpu.sync_copy(scores_hbm.at[pl.ds(sid * CH, CH)], s_v)

        for j in range(NV):
            b = lax.bitcast_convert_type(s_v[pl.ds(j * 16, 16)], jnp.int32)
            m_v[pl.ds(j * 16, 16)] = b ^ ((b >> 31) & jnp.int32(0x7FFFFFFF))

        k_rem = jnp.int32(K_MASK)
        pval = jnp.int32(0)
        for r in range(4):
            shift = 24 - 8 * r

            def _zero(j, _):
                for k in range(8):
                    h2d[pl.ds(j * 128 + k * 16, 16)] = jnp.zeros(
                        (16,), jnp.int32)
                return 0
            lax.fori_loop(0, 32, _zero, 0)

            def _hist(j, _):
                m = m_v[pl.ds(j * 16, 16)]
                if r == 0:
                    byte = (m >> 24) + 128
                    plsc.addupdate_scatter(h2d, [lane * 256 + byte], ones_i)
                else:
                    byte = (m >> shift) & 255
                    pm = (m >> (shift + 8)) == pval
                    plsc.addupdate_scatter(
                        h2d, [lane * 256 + byte], ones_i, mask=pm)
                return 0
            lax.fori_loop(0, NV, _hist, 0)

            def _lmerge(j, _):
                a = h2d[pl.ds(j * 16, 16)]
                for l in range(1, 16):
                    a = a + h2d[pl.ds(l * 256 + j * 16, 16)]
                gh[pl.ds(j * 16, 16)] = a
                return 0
            lax.fori_loop(0, 16, _lmerge, 0)

            half = (r % 2) * 4096
            pltpu.sync_copy(gh, sh_hist.at[pl.ds(half + sid * 256, 256)])
            plsc.subcore_barrier()
            pltpu.sync_copy(sh_hist.at[pl.ds(half, 4096)], h2d)
            lax.fori_loop(0, 16, _lmerge, 0)

            def _scan(i, c):
                carry, best = c
                ch = 15 - i
                chunk = gh[pl.ds(ch * 16, 16)]
                suff = jnp.flip(plsc.cumsum(jnp.flip(chunk, 0)), 0) + carry
                bidx = ch * 16 + lane
                cand = jnp.max(jnp.where(suff >= k_rem, bidx, -1))
                return (carry + jnp.sum(chunk), jnp.maximum(best, cand))
            _, bstar = lax.fori_loop(
                0, 16, _scan, (jnp.int32(0), jnp.int32(-1)))

            def _pick(i, c):
                carry, cge, hb = c
                ch = 15 - i
                chunk = gh[pl.ds(ch * 16, 16)]
                suff = jnp.flip(plsc.cumsum(jnp.flip(chunk, 0)), 0) + carry
                sel = (ch * 16 + lane) == bstar
                cge = jnp.maximum(cge, jnp.max(jnp.where(sel, suff, -1)))
                hb = jnp.maximum(hb, jnp.max(jnp.where(sel, chunk, -1)))
                return (carry + jnp.sum(chunk), cge, hb)
            _, cnt_ge, hist_b = lax.fori_loop(
                0, 16, _pick, (jnp.int32(0), jnp.int32(-1), jnp.int32(-1)))

            k_rem = k_rem - (cnt_ge - hist_b)
            pval = (bstar - 128) if r == 0 else ((pval << 8) | bstar)

        vstar = pval
        t_ties = k_rem

        counts = plsc.load_gather(h2d, [lane * 256 + (vstar & 255)])
        n_eq = jnp.sum(counts)
        keep_cnt = n_eq - t_ties
        pref = plsc.cumsum(counts) - counts
        quota = keep_cnt - jnp.sum(jnp.where(lane == sid, pref, 0))

        def _mark(j, c):
            run, mx = c
            m = m_v[pl.ds(j * 16, 16)]
            s = s_v[pl.ds(j * 16, 16)]
            eq = m == vstar
            eqi = eq.astype(jnp.int32)
            rank = plsc.cumsum(eqi) + run
            keep = (m < vstar) | (eq & (rank <= quota))
            m_v[pl.ds(j * 16, 16)] = keep.astype(jnp.int32)
            mx = jnp.maximum(mx, jnp.where(keep, s, -jnp.inf))
            return (run + jnp.sum(eqi), mx)
        _, mxv = lax.fori_loop(
            0, NV, _mark, (jnp.int32(0), jnp.full((16,), -jnp.inf)))
        mx_t = jnp.max(mxv)

        msafe = jnp.maximum(mx_t, jnp.float32(-1e30))

        def _lsum(j, a):
            kp = m_v[pl.ds(j * 16, 16)] == 1
            return a + jnp.where(
                kp, jnp.exp(s_v[pl.ds(j * 16, 16)] - msafe), 0.0)
        s_t = jnp.sum(lax.fori_loop(
            0, NV, _lsum, jnp.zeros((16,), jnp.float32)))

        t16f[...] = jnp.zeros((16,), jnp.float32) + mx_t
        pltpu.sync_copy(t16f, sh_f.at[pl.ds(sid * 16, 16)])
        t16f[...] = jnp.zeros((16,), jnp.float32) + s_t
        pltpu.sync_copy(t16f, sh_f.at[pl.ds(256 + sid * 16, 16)])
        plsc.subcore_barrier()
        pltpu.sync_copy(sh_f, ghf)
        mxs = plsc.load_gather(ghf, [lane * 16])
        sms = plsc.load_gather(ghf, [lane * 16 + 256])
        gmax = jnp.max(mxs)
        z = jnp.sum(sms * jnp.exp(mxs - gmax))

        def _fin(j, _):
            kp = m_v[pl.ds(j * 16, 16)] == 1
            e_v[pl.ds(j * 16, 16)] = jnp.where(
                kp, jnp.exp(s_v[pl.ds(j * 16, 16)] - gmax) / z, 0.0)
            return 0
        lax.fori_loop(0, NV, _fin, 0)
        pltpu.sync_copy(e_v, w_hbm.at[pl.ds(sid * CH, CH)])


_select_sc = functools.partial(
    pl.kernel,
    out_type=jax.ShapeDtypeStruct((N,), jnp.float32),
    mesh=plsc.VectorSubcoreMesh(core_axis_name="c", subcore_axis_name="s",
                                num_cores=2, num_subcores=16),
    compiler_params=pltpu.CompilerParams(needs_layout_passes=False),
    scratch_types=[
        pltpu.VMEM((CH,), jnp.float32),
        pltpu.VMEM((CH,), jnp.int32),
        pltpu.VMEM((4096,), jnp.int32),
        pltpu.VMEM((256,), jnp.int32),
        pltpu.VMEM((512,), jnp.float32),
        pltpu.VMEM((CH,), jnp.float32),
        pltpu.VMEM((16,), jnp.float32),
        pltpu.VMEM_SHARED((8192,), jnp.int32),
        pltpu.VMEM_SHARED((512,), jnp.float32),
    ],
)(_stage_b)


def _stage_c(w_ref, feat_ref, wp_ref, bp_ref, out_ref, acc_ref):
    i = pl.program_id(0)

    @pl.when(i == 0)
    def _init():
        acc_ref[...] = jnp.zeros((1, D), jnp.float32)

    wv = w_ref[...].reshape(1, w_ref.shape[0]).astype(jnp.bfloat16)
    acc_ref[...] += lax.dot_general(
        wv, feat_ref[...],
        (((1,), (0,)), ((), ())), preferred_element_type=jnp.float32)

    @pl.when(i == pl.num_programs(0) - 1)
    def _fin():
        out_ref[...] = lax.dot_general(
            acc_ref[...], wp_ref[...], (((1,), (0,)), ((), ()))) + bp_ref[...]


@jax.jit
def kernel(x, W1, b1, Va, ba, wa, bwa, Wp, bp):
    x2 = x.reshape(N, D_IN)
    feat, scores = pl.pallas_call(
        _stage_a,
        grid=(T,),
        in_specs=[
            pl.BlockSpec((R, D_IN), lambda i: (i, 0)),
            pl.BlockSpec((D_IN, D), lambda i: (0, 0)),
            pl.BlockSpec((1, D), lambda i: (0, 0)),
            pl.BlockSpec((D, DA), lambda i: (0, 0)),
            pl.BlockSpec((1, DA), lambda i: (0, 0)),
            pl.BlockSpec((DA, 1), lambda i: (0, 0)),
            pl.BlockSpec((1, 1), lambda i: (0, 0)),
        ],
        out_specs=[
            pl.BlockSpec((R, D), lambda i: (i, 0)),
            pl.BlockSpec((R,), lambda i: (i,)),
        ],
        out_shape=[
            jax.ShapeDtypeStruct((N, D), jnp.bfloat16),
            jax.ShapeDtypeStruct((N,), jnp.float32),
        ],
    )(x2, W1, b1.reshape(1, D), Va, ba.reshape(1, DA), wa,
      bwa.reshape(1, 1))

    w = _select_sc(scores)

    out = pl.pallas_call(
        _stage_c,
        grid=(8,),
        in_specs=[
            pl.BlockSpec((N // 8,), lambda i: (i,)),
            pl.BlockSpec((N // 8, D), lambda i: (i, 0)),
            pl.BlockSpec((D, 2), lambda i: (0, 0)),
            pl.BlockSpec((1, 2), lambda i: (0, 0)),
        ],
        out_specs=pl.BlockSpec((1, 2), lambda i: (0, 0)),
        out_shape=jax.ShapeDtypeStruct((1, 2), jnp.float32),
        scratch_shapes=[pltpu.VMEM((1, D), jnp.float32)],
    )(w, feat, Wp, bp.reshape(1, 2))
    return out


# speedup vs baseline: 1.1095x; 1.0218x over previous
import functools

import jax
import jax.numpy as jnp
from jax import lax
from jax.experimental import pallas as pl
from jax.experimental.pallas import tpu as pltpu
from jax.experimental.pallas import tpu_sc as plsc

N = 8192
D_IN = 1024
D = 512
DA = 128
K_MASK = 819
R = 2048
T = N // R

NT = 16
CH = N // NT
NV = CH // 16


def _stage_a(x_ref, w1_ref, b1_ref, va_ref, ba_ref, wa_ref, bwa_ref,
             feat_ref, s_ref):
    xt = x_ref[...]
    feat = jnp.maximum(
        lax.dot_general(xt, w1_ref[...], (((1,), (0,)), ((), ()))) +
        b1_ref[...], 0.0)
    feat_ref[...] = feat.astype(jnp.bfloat16)
    h = jax.nn.gelu(
        lax.dot_general(feat, va_ref[...], (((1,), (0,)), ((), ()))) +
        ba_ref[...])
    s = lax.dot_general(
        wa_ref[...], h, (((0,), (1,)), ((), ()))) + bwa_ref[0, 0]
    s_ref[...] = s.reshape(s_ref.shape)


def _stage_b(scores_hbm, w_hbm, s_v, m_v, h2d, gh, ghf, e_v, t16f,
             sh_hist, sh_f):
    cid = lax.axis_index("c")
    sid = lax.axis_index("s")

    @pl.when(cid == 0)
    def _work():
        lane = lax.iota(jnp.int32, 16)
        ones_i = jnp.ones((16,), jnp.int32)

        pltpu.sync_copy(scores_hbm.at[pl.ds(sid * CH, CH)], s_v)

        for j in range(NV):
            b = lax.bitcast_convert_type(s_v[pl.ds(j * 16, 16)], jnp.int32)
            m_v[pl.ds(j * 16, 16)] = b ^ ((b >> 31) & jnp.int32(0x7FFFFFFF))

        k_rem = jnp.int32(K_MASK)
        pval = jnp.int32(0)
        for r in range(4):
            shift = 24 - 8 * r

            def _zero(j, _):
                for k in range(8):
                    h2d[pl.ds(j * 128 + k * 16, 16)] = jnp.zeros(
                        (16,), jnp.int32)
                return 0
            lax.fori_loop(0, 32, _zero, 0)

            def _hist(j, _):
                m = m_v[pl.ds(j * 16, 16)]
                if r == 0:
                    byte = (m >> 24) + 128
                    plsc.addupdate_scatter(h2d, [lane * 256 + byte], ones_i)
                else:
                    byte = (m >> shift) & 255
                    pm = (m >> (shift + 8)) == pval
                    plsc.addupdate_scatter(
                        h2d, [lane * 256 + byte], ones_i, mask=pm)
                return 0
            lax.fori_loop(0, NV, _hist, 0)

            def _lmerge(j, _):
                a = h2d[pl.ds(j * 16, 16)]
                for l in range(1, 16):
                    a = a + h2d[pl.ds(l * 256 + j * 16, 16)]
                gh[pl.ds(j * 16, 16)] = a
                return 0
            lax.fori_loop(0, 16, _lmerge, 0)

            half = (r % 2) * 4096
            pltpu.sync_copy(gh, sh_hist.at[pl.ds(half + sid * 256, 256)])
            plsc.subcore_barrier()
            pltpu.sync_copy(sh_hist.at[pl.ds(half, 4096)], h2d)
            lax.fori_loop(0, 16, _lmerge, 0)

            def _scan(i, c):
                carry, best = c
                ch = 15 - i
                chunk = gh[pl.ds(ch * 16, 16)]
                suff = jnp.flip(plsc.cumsum(jnp.flip(chunk, 0)), 0) + carry
                bidx = ch * 16 + lane
                cand = jnp.max(jnp.where(suff >= k_rem, bidx, -1))
                return (carry + jnp.sum(chunk), jnp.maximum(best, cand))
            _, bstar = lax.fori_loop(
                0, 16, _scan, (jnp.int32(0), jnp.int32(-1)))

            def _pick(i, c):
                carry, cge, hb = c
                ch = 15 - i
                chunk = gh[pl.ds(ch * 16, 16)]
                suff = jnp.flip(plsc.cumsum(jnp.flip(chunk, 0)), 0) + carry
                sel = (ch * 16 + lane) == bstar
                cge = jnp.maximum(cge, jnp.max(jnp.where(sel, suff, -1)))
                hb = jnp.maximum(hb, jnp.max(jnp.where(sel, chunk, -1)))
                return (carry + jnp.sum(chunk), cge, hb)
            _, cnt_ge, hist_b = lax.fori_loop(
                0, 16, _pick, (jnp.int32(0), jnp.int32(-1), jnp.int32(-1)))

            k_rem = k_rem - (cnt_ge - hist_b)
            pval = (bstar - 128) if r == 0 else ((pval << 8) | bstar)

        vstar = pval
        t_ties = k_rem

        counts = plsc.load_gather(h2d, [lane * 256 + (vstar & 255)])
        n_eq = jnp.sum(counts)
        keep_cnt = n_eq - t_ties
        pref = plsc.cumsum(counts) - counts
        quota = keep_cnt - jnp.sum(jnp.where(lane == sid, pref, 0))

        def _mark(j, c):
            run, mx = c
            m = m_v[pl.ds(j * 16, 16)]
            s = s_v[pl.ds(j * 16, 16)]
            eq = m == vstar
            eqi = eq.astype(jnp.int32)
            rank = plsc.cumsum(eqi) + run
            keep = (m < vstar) | (eq & (rank <= quota))
            m_v[pl.ds(j * 16, 16)] = keep.astype(jnp.int32)
            mx = jnp.maximum(mx, jnp.where(keep, s, -jnp.inf))
            return (run + jnp.sum(eqi), mx)
        _, mxv = lax.fori_loop(
            0, NV, _mark, (jnp.int32(0), jnp.full((16,), -jnp.inf)))
        mx_t = jnp.max(mxv)

        msafe = jnp.maximum(mx_t, jnp.float32(-1e30))

        def _lsum(j, a):
            kp = m_v[pl.ds(j * 16, 16)] == 1
            return a + jnp.where(
                kp, jnp.exp(s_v[pl.ds(j * 16, 16)] - msafe), 0.0)
        s_t = jnp.sum(lax.fori_loop(
            0, NV, _lsum, jnp.zeros((16,), jnp.float32)))

        t16f[...] = jnp.zeros((16,), jnp.float32) + mx_t
        pltpu.sync_copy(t16f, sh_f.at[pl.ds(sid * 16, 16)])
        t16f[...] = jnp.zeros((16,), jnp.float32) + s_t
        pltpu.sync_copy(t16f, sh_f.at[pl.ds(256 + sid * 16, 16)])
        plsc.subcore_barrier()
        pltpu.sync_copy(sh_f, ghf)
        mxs = plsc.load_gather(ghf, [lane * 16])
        sms = plsc.load_gather(ghf, [lane * 16 + 256])
        gmax = jnp.max(mxs)
        z = jnp.sum(sms * jnp.exp(mxs - gmax))

        def _fin(j, _):
            kp = m_v[pl.ds(j * 16, 16)] == 1
            e_v[pl.ds(j * 16, 16)] = jnp.where(
                kp, jnp.exp(s_v[pl.ds(j * 16, 16)] - gmax) / z, 0.0)
            return 0
        lax.fori_loop(0, NV, _fin, 0)
        pltpu.sync_copy(e_v, w_hbm.at[pl.ds(sid * CH, CH)])


_select_sc = functools.partial(
    pl.kernel,
    out_type=jax.ShapeDtypeStruct((N,), jnp.float32),
    mesh=plsc.VectorSubcoreMesh(core_axis_name="c", subcore_axis_name="s",
                                num_cores=2, num_subcores=16),
    compiler_params=pltpu.CompilerParams(needs_layout_passes=False),
    scratch_types=[
        pltpu.VMEM((CH,), jnp.float32),
        pltpu.VMEM((CH,), jnp.int32),
        pltpu.VMEM((4096,), jnp.int32),
        pltpu.VMEM((256,), jnp.int32),
        pltpu.VMEM((512,), jnp.float32),
        pltpu.VMEM((CH,), jnp.float32),
        pltpu.VMEM((16,), jnp.float32),
        pltpu.VMEM_SHARED((8192,), jnp.int32),
        pltpu.VMEM_SHARED((512,), jnp.float32),
    ],
)(_stage_b)


def _stage_c(w_ref, feat_ref, wp_ref, bp_ref, out_ref, acc_ref):
    i = pl.program_id(0)

    @pl.when(i == 0)
    def _init():
        acc_ref[...] = jnp.zeros((1, D), jnp.float32)

    wv = w_ref[...].reshape(1, w_ref.shape[0]).astype(jnp.bfloat16)
    acc_ref[...] += lax.dot_general(
        wv, feat_ref[...],
        (((1,), (0,)), ((), ())), preferred_element_type=jnp.float32)

    @pl.when(i == pl.num_programs(0) - 1)
    def _fin():
        out_ref[...] = lax.dot_general(
            acc_ref[...], wp_ref[...], (((1,), (0,)), ((), ()))) + bp_ref[...]


@jax.jit
def kernel(x, W1, b1, Va, ba, wa, bwa, Wp, bp):
    x2 = x.reshape(N, D_IN)
    feat, scores = pl.pallas_call(
        _stage_a,
        grid=(T,),
        in_specs=[
            pl.BlockSpec((R, D_IN), lambda i: (i, 0)),
            pl.BlockSpec((D_IN, D), lambda i: (0, 0)),
            pl.BlockSpec((1, D), lambda i: (0, 0)),
            pl.BlockSpec((D, DA), lambda i: (0, 0)),
            pl.BlockSpec((1, DA), lambda i: (0, 0)),
            pl.BlockSpec((DA, 1), lambda i: (0, 0)),
            pl.BlockSpec((1, 1), lambda i: (0, 0)),
        ],
        out_specs=[
            pl.BlockSpec((R, D), lambda i: (i, 0)),
            pl.BlockSpec((R,), lambda i: (i,)),
        ],
        out_shape=[
            jax.ShapeDtypeStruct((N, D), jnp.bfloat16),
            jax.ShapeDtypeStruct((N,), jnp.float32),
        ],
    )(x2, W1, b1.reshape(1, D), Va, ba.reshape(1, DA), wa,
      bwa.reshape(1, 1))

    w = _select_sc(scores)

    out = pl.pallas_call(
        _stage_c,
        grid=(8,),
        in_specs=[
            pl.BlockSpec((N // 8,), lambda i: (i,)),
            pl.BlockSpec((N // 8, D), lambda i: (i, 0)),
            pl.BlockSpec((D, 2), lambda i: (0, 0)),
            pl.BlockSpec((1, 2), lambda i: (0, 0)),
        ],
        out_specs=pl.BlockSpec((1, 2), lambda i: (0, 0)),
        out_shape=jax.ShapeDtypeStruct((1, 2), jnp.float32),
        scratch_shapes=[pltpu.VMEM((1, D), jnp.float32)],
    )(w, feat, Wp, bp.reshape(1, 2))
    return out


# speedup vs baseline: 1.1232x; 1.0124x over previous
import functools

import jax
import jax.numpy as jnp
from jax import lax
from jax.experimental import pallas as pl
from jax.experimental.pallas import tpu as pltpu
from jax.experimental.pallas import tpu_sc as plsc

N = 8192
D_IN = 1024
D = 512
DA = 128
K_MASK = 819
R = 4096
T = N // R

NT = 16
CH = N // NT
NV = CH // 16


def _stage_a(x_ref, w1_ref, b1_ref, va_ref, ba_ref, wa_ref, bwa_ref,
             feat_ref, s_ref):
    xt = x_ref[...]
    feat = jnp.maximum(
        lax.dot_general(xt, w1_ref[...], (((1,), (0,)), ((), ()))) +
        b1_ref[...], 0.0)
    feat_ref[...] = feat.astype(jnp.bfloat16)
    h = jax.nn.gelu(
        lax.dot_general(feat, va_ref[...], (((1,), (0,)), ((), ()))) +
        ba_ref[...])
    s = lax.dot_general(
        wa_ref[...], h, (((0,), (1,)), ((), ()))) + bwa_ref[0, 0]
    s_ref[...] = s.reshape(s_ref.shape)


def _stage_b(scores_hbm, w_hbm, s_v, m_v, h2d, gh, ghf, e_v, t16f,
             sh_hist, sh_f):
    cid = lax.axis_index("c")
    sid = lax.axis_index("s")

    @pl.when(cid == 0)
    def _work():
        lane = lax.iota(jnp.int32, 16)
        ones_i = jnp.ones((16,), jnp.int32)

        pltpu.sync_copy(scores_hbm.at[pl.ds(sid * CH, CH)], s_v)

        for j in range(NV):
            b = lax.bitcast_convert_type(s_v[pl.ds(j * 16, 16)], jnp.int32)
            m_v[pl.ds(j * 16, 16)] = b ^ ((b >> 31) & jnp.int32(0x7FFFFFFF))

        k_rem = jnp.int32(K_MASK)
        pval = jnp.int32(0)
        for r in range(4):
            shift = 24 - 8 * r

            def _zero(j, _):
                for k in range(8):
                    h2d[pl.ds(j * 128 + k * 16, 16)] = jnp.zeros(
                        (16,), jnp.int32)
                return 0
            lax.fori_loop(0, 32, _zero, 0)

            def _hist(j, _):
                m = m_v[pl.ds(j * 16, 16)]
                if r == 0:
                    byte = (m >> 24) + 128
                    plsc.addupdate_scatter(h2d, [lane * 256 + byte], ones_i)
                else:
                    byte = (m >> shift) & 255
                    pm = (m >> (shift + 8)) == pval
                    plsc.addupdate_scatter(
                        h2d, [lane * 256 + byte], ones_i, mask=pm)
                return 0
            lax.fori_loop(0, NV, _hist, 0)

            def _lmerge(j, _):
                a = h2d[pl.ds(j * 16, 16)]
                for l in range(1, 16):
                    a = a + h2d[pl.ds(l * 256 + j * 16, 16)]
                gh[pl.ds(j * 16, 16)] = a
                return 0
            lax.fori_loop(0, 16, _lmerge, 0)

            half = (r % 2) * 4096
            pltpu.sync_copy(gh, sh_hist.at[pl.ds(half + sid * 256, 256)])
            plsc.subcore_barrier()
            pltpu.sync_copy(sh_hist.at[pl.ds(half, 4096)], h2d)
            lax.fori_loop(0, 16, _lmerge, 0)

            def _scan(i, c):
                carry, best = c
                ch = 15 - i
                chunk = gh[pl.ds(ch * 16, 16)]
                suff = jnp.flip(plsc.cumsum(jnp.flip(chunk, 0)), 0) + carry
                bidx = ch * 16 + lane
                cand = jnp.max(jnp.where(suff >= k_rem, bidx, -1))
                return (carry + jnp.sum(chunk), jnp.maximum(best, cand))
            _, bstar = lax.fori_loop(
                0, 16, _scan, (jnp.int32(0), jnp.int32(-1)))

            def _pick(i, c):
                carry, cge, hb = c
                ch = 15 - i
                chunk = gh[pl.ds(ch * 16, 16)]
                suff = jnp.flip(plsc.cumsum(jnp.flip(chunk, 0)), 0) + carry
                sel = (ch * 16 + lane) == bstar
                cge = jnp.maximum(cge, jnp.max(jnp.where(sel, suff, -1)))
                hb = jnp.maximum(hb, jnp.max(jnp.where(sel, chunk, -1)))
                return (carry + jnp.sum(chunk), cge, hb)
            _, cnt_ge, hist_b = lax.fori_loop(
                0, 16, _pick, (jnp.int32(0), jnp.int32(-1), jnp.int32(-1)))

            k_rem = k_rem - (cnt_ge - hist_b)
            pval = (bstar - 128) if r == 0 else ((pval << 8) | bstar)

        vstar = pval
        t_ties = k_rem

        counts = plsc.load_gather(h2d, [lane * 256 + (vstar & 255)])
        n_eq = jnp.sum(counts)
        keep_cnt = n_eq - t_ties
        pref = plsc.cumsum(counts) - counts
        quota = keep_cnt - jnp.sum(jnp.where(lane == sid, pref, 0))

        def _mark(j, c):
            run, mx = c
            m = m_v[pl.ds(j * 16, 16)]
            s = s_v[pl.ds(j * 16, 16)]
            eq = m == vstar
            eqi = eq.astype(jnp.int32)
            rank = plsc.cumsum(eqi) + run
            keep = (m < vstar) | (eq & (rank <= quota))
            m_v[pl.ds(j * 16, 16)] = keep.astype(jnp.int32)
            mx = jnp.maximum(mx, jnp.where(keep, s, -jnp.inf))
            return (run + jnp.sum(eqi), mx)
        _, mxv = lax.fori_loop(
            0, NV, _mark, (jnp.int32(0), jnp.full((16,), -jnp.inf)))
        mx_t = jnp.max(mxv)

        msafe = jnp.maximum(mx_t, jnp.float32(-1e30))

        def _lsum(j, a):
            kp = m_v[pl.ds(j * 16, 16)] == 1
            return a + jnp.where(
                kp, jnp.exp(s_v[pl.ds(j * 16, 16)] - msafe), 0.0)
        s_t = jnp.sum(lax.fori_loop(
            0, NV, _lsum, jnp.zeros((16,), jnp.float32)))

        t16f[...] = jnp.zeros((16,), jnp.float32) + mx_t
        pltpu.sync_copy(t16f, sh_f.at[pl.ds(sid * 16, 16)])
        t16f[...] = jnp.zeros((16,), jnp.float32) + s_t
        pltpu.sync_copy(t16f, sh_f.at[pl.ds(256 + sid * 16, 16)])
        plsc.subcore_barrier()
        pltpu.sync_copy(sh_f, ghf)
        mxs = plsc.load_gather(ghf, [lane * 16])
        sms = plsc.load_gather(ghf, [lane * 16 + 256])
        gmax = jnp.max(mxs)
        z = jnp.sum(sms * jnp.exp(mxs - gmax))

        def _fin(j, _):
            kp = m_v[pl.ds(j * 16, 16)] == 1
            e_v[pl.ds(j * 16, 16)] = jnp.where(
                kp, jnp.exp(s_v[pl.ds(j * 16, 16)] - gmax) / z, 0.0)
            return 0
        lax.fori_loop(0, NV, _fin, 0)
        pltpu.sync_copy(e_v, w_hbm.at[pl.ds(sid * CH, CH)])


_select_sc = functools.partial(
    pl.kernel,
    out_type=jax.ShapeDtypeStruct((N,), jnp.float32),
    mesh=plsc.VectorSubcoreMesh(core_axis_name="c", subcore_axis_name="s",
                                num_cores=2, num_subcores=16),
    compiler_params=pltpu.CompilerParams(needs_layout_passes=False),
    scratch_types=[
        pltpu.VMEM((CH,), jnp.float32),
        pltpu.VMEM((CH,), jnp.int32),
        pltpu.VMEM((4096,), jnp.int32),
        pltpu.VMEM((256,), jnp.int32),
        pltpu.VMEM((512,), jnp.float32),
        pltpu.VMEM((CH,), jnp.float32),
        pltpu.VMEM((16,), jnp.float32),
        pltpu.VMEM_SHARED((8192,), jnp.int32),
        pltpu.VMEM_SHARED((512,), jnp.float32),
    ],
)(_stage_b)


def _stage_c(w_ref, feat_ref, wp_ref, bp_ref, out_ref, acc_ref):
    i = pl.program_id(0)

    @pl.when(i == 0)
    def _init():
        acc_ref[...] = jnp.zeros((1, D), jnp.float32)

    wv = w_ref[...].reshape(1, w_ref.shape[0]).astype(jnp.bfloat16)
    acc_ref[...] += lax.dot_general(
        wv, feat_ref[...],
        (((1,), (0,)), ((), ())), preferred_element_type=jnp.float32)

    @pl.when(i == pl.num_programs(0) - 1)
    def _fin():
        out_ref[...] = lax.dot_general(
            acc_ref[...], wp_ref[...], (((1,), (0,)), ((), ()))) + bp_ref[...]


@jax.jit
def kernel(x, W1, b1, Va, ba, wa, bwa, Wp, bp):
    x2 = x.reshape(N, D_IN)
    feat, scores = pl.pallas_call(
        _stage_a,
        grid=(T,),
        in_specs=[
            pl.BlockSpec((R, D_IN), lambda i: (i, 0)),
            pl.BlockSpec((D_IN, D), lambda i: (0, 0)),
            pl.BlockSpec((1, D), lambda i: (0, 0)),
            pl.BlockSpec((D, DA), lambda i: (0, 0)),
            pl.BlockSpec((1, DA), lambda i: (0, 0)),
            pl.BlockSpec((DA, 1), lambda i: (0, 0)),
            pl.BlockSpec((1, 1), lambda i: (0, 0)),
        ],
        out_specs=[
            pl.BlockSpec((R, D), lambda i: (i, 0)),
            pl.BlockSpec((R,), lambda i: (i,)),
        ],
        out_shape=[
            jax.ShapeDtypeStruct((N, D), jnp.bfloat16),
            jax.ShapeDtypeStruct((N,), jnp.float32),
        ],
    )(x2, W1, b1.reshape(1, D), Va, ba.reshape(1, DA), wa,
      bwa.reshape(1, 1))

    w = _select_sc(scores)

    out = pl.pallas_call(
        _stage_c,
        grid=(4,),
        in_specs=[
            pl.BlockSpec((N // 4,), lambda i: (i,)),
            pl.BlockSpec((N // 4, D), lambda i: (i, 0)),
            pl.BlockSpec((D, 2), lambda i: (0, 0)),
            pl.BlockSpec((1, 2), lambda i: (0, 0)),
        ],
        out_specs=pl.BlockSpec((1, 2), lambda i: (0, 0)),
        out_shape=jax.ShapeDtypeStruct((1, 2), jnp.float32),
        scratch_shapes=[pltpu.VMEM((1, D), jnp.float32)],
    )(w, feat, Wp, bp.reshape(1, 2))
    return out
